# baseline (speedup 1.0000x reference)
import functools

import jax
import jax.numpy as jnp
from jax import lax
from jax.experimental import pallas as pl
from jax.experimental.pallas import tpu as pltpu
from jax.experimental.pallas import tpu_sc as plsc

EMB = 64
NC = 2
NS = 16
NW = NC * NS
BL = 128


def _relu_t_body(w_ref, o_ref):
    o_ref[...] = jnp.maximum(w_ref[...], 0.0).T


def _relu_t_table(w):
    return pl.pallas_call(
        _relu_t_body,
        out_shape=jax.ShapeDtypeStruct((w.shape[1], w.shape[0]), w.dtype),
    )(w)


def _pack_bf16_pairs(tt):
    lo = tt[0::2].astype(jnp.bfloat16).astype(jnp.float32)
    hi = tt[1::2].astype(jnp.bfloat16).astype(jnp.float32)
    lo_u = lax.shift_right_logical(lax.bitcast_convert_type(lo, jnp.uint32), jnp.uint32(16))
    hi_u = lax.bitcast_convert_type(hi, jnp.uint32) & jnp.uint32(0xFFFF0000)
    return lax.bitcast_convert_type(lo_u | hi_u, jnp.int32)


def _make_gather(nb, nh, vocab):
    nbt = nb // BL
    assert nbt == NW and nh % 2 == 0
    mesh = plsc.VectorSubcoreMesh(core_axis_name="c", subcore_axis_name="s")

    @functools.partial(
        pl.kernel,
        mesh=mesh,
        compiler_params=pltpu.CompilerParams(
            use_tc_tiling_on_sc=False, needs_layout_passes=False),
        out_type=jax.ShapeDtypeStruct((nh, 8, nbt, 8, BL), jnp.float32),
        scratch_types=(
            [pltpu.VMEM((nh, BL), jnp.int32),
             pltpu.VMEM((EMB // 2 * vocab,), jnp.int32)]
            + [pltpu.VMEM((8, 8, 129), jnp.float32) for _ in range(2)]
            + [pltpu.SemaphoreType.DMA for _ in range(2)]
        ),
    )
    def gather_kernel(xt_hbm, tab_hbm, out_hbm, idx_v, tab_v,
                      tb0, tb1, os0, os1):
        tb = (tb0, tb1)
        osem = (os0, os1)

        wid = lax.axis_index("s") * NC + lax.axis_index("c")

        pltpu.sync_copy(xt_hbm.at[:, pl.ds(wid * BL, BL)], idx_v)
        pltpu.sync_copy(tab_hbm, tab_v)

        def start_write(h, s):
            pltpu.make_async_copy(
                tb[s].at[:, :, pl.ds(0, BL)], out_hbm.at[h, :, wid],
                osem[s]).start()

        def wait_write(s):
            pltpu.make_async_copy(
                tb[s].at[:, :, pl.ds(0, BL)], out_hbm.at[0, :, wid],
                osem[s]).wait()

        mask_hi = jnp.full((16,), -65536, jnp.int32)

        def lookup(h, s):
            idxvs = tuple(
                idx_v[h, pl.ds(c * 16, 16)] for c in range(8))

            def j_body(j, addrs):
                for u in range(8):
                    e2 = j * 8 + u
                    et_c = u >> 2
                    es = (2 * u) & 7
                    vs = [plsc.load_gather(tab_v, [a]) for a in addrs]
                    for c, v in enumerate(vs):
                        lo = plsc.bitcast(
                            lax.shift_left(v, jnp.full((16,), 16, jnp.int32)),
                            jnp.float32)
                        hi = plsc.bitcast(v & mask_hi, jnp.float32)
                        tb[s][2 * j + et_c, es, pl.ds(c * 16, 16)] = lo
                        tb[s][2 * j + et_c, es + 1, pl.ds(c * 16, 16)] = hi
                    addrs = tuple(a + vocab for a in addrs)
                return addrs

            lax.fori_loop(0, 4, j_body, idxvs)

        def group_body(hh, _):
            for s in range(2):
                h = hh * 2 + s

                @pl.when(hh >= 1)
                def _():
                    wait_write(s)

                lookup(h, s)
                start_write(h, s)

            return 0

        lax.fori_loop(0, nh // 2, group_body, 0)
        wait_write(0)
        wait_write(1)

    return gather_kernel


def kernel(x, embedding_weight):
    nb, nh = x.shape
    vocab = embedding_weight.shape[0]
    table_t = _relu_t_table(embedding_weight)
    packed = _pack_bf16_pairs(table_t).reshape(-1)
    xt = x.T
    y = _make_gather(nb, nh, vocab)(xt, packed)
    return y.transpose(2, 4, 0, 1, 3).reshape(nb, nh, EMB)
